# baseline (speedup 1.0000x reference)
import jax
import jax.numpy as jnp
from jax.experimental import pallas as pl
from jax.experimental.pallas import tpu as pltpu

_N = 4096
_D = 256
_K = 10
_R = 512


def _proj_kernel(x_ref, w_ref, o_ref):
    o_ref[:, :] = jax.lax.dot(
        x_ref[:, :].astype(jnp.bfloat16), w_ref[:, :].astype(jnp.bfloat16),
        preferred_element_type=jnp.float32,
    )


def _knn_kernel(t_ref, xw_blk_ref, xw_ref, sqr_ref, sql_ref, idx_ref, val_ref):
    t = t_ref[0]
    xw_b = xw_ref[:, :].astype(jnp.bfloat16)
    xw_blk_b = xw_blk_ref[:, :].astype(jnp.bfloat16)
    g = jax.lax.dot_general(
        xw_blk_b, xw_b,
        (((1,), (1,)), ((), ())),
        preferred_element_type=jnp.float32,
    )
    lq = (sqr_ref[:, :] + sql_ref[:, :] - 2.0 * g) * t
    iota_f = jax.lax.broadcasted_iota(jnp.int32, (_R, _N), 1).astype(jnp.float32)
    for k in range(_K):
        m = jnp.min(lq, axis=1)
        mask = lq <= m[:, None]
        sel = jnp.where(mask, iota_f, jnp.float32(_N))
        jf = jnp.min(sel, axis=1)
        idx_ref[:, k] = jf.astype(jnp.int32)
        val_ref[:, k] = -m
        lq = jnp.where(sel <= jf[:, None], jnp.float32(jnp.inf), lq)


@jax.jit
def kernel(x, A, W, temperature):
    del A
    t = jnp.exp(jnp.clip(temperature, -5.0, 5.0)).astype(jnp.float32)

    xw = pl.pallas_call(
        _proj_kernel,
        grid=(_N // _R,),
        in_specs=[
            pl.BlockSpec((_R, _D), lambda i: (i, 0)),
            pl.BlockSpec((_D, _D), lambda i: (0, 0)),
        ],
        out_specs=pl.BlockSpec((_R, _D), lambda i: (i, 0)),
        out_shape=jax.ShapeDtypeStruct((_N, _D), jnp.float32),
    )(x, W)

    sq = jnp.sum(xw * xw, axis=-1)

    idx, val = pl.pallas_call(
        _knn_kernel,
        grid=(_N // _R,),
        in_specs=[
            pl.BlockSpec(memory_space=pltpu.SMEM),
            pl.BlockSpec((_R, _D), lambda i: (i, 0)),
            pl.BlockSpec((_N, _D), lambda i: (0, 0)),
            pl.BlockSpec((_R, 1), lambda i: (i, 0)),
            pl.BlockSpec((1, _N), lambda i: (0, 0)),
        ],
        out_specs=[
            pl.BlockSpec((_R, _K), lambda i: (i, 0)),
            pl.BlockSpec((_R, _K), lambda i: (i, 0)),
        ],
        out_shape=[
            jax.ShapeDtypeStruct((_N, _K), jnp.int32),
            jax.ShapeDtypeStruct((_N, _K), jnp.float32),
        ],
    )(t.reshape(1), xw, xw, sq[:, None], sq[None, :])

    rows = jnp.broadcast_to(
        jnp.arange(_N, dtype=jnp.int32)[:, None], (_N, _K)
    ).reshape(-1)
    edges_hat = jnp.stack((idx.reshape(-1), rows), axis=0)
    return (xw[None], edges_hat, val[None])


# speedup vs baseline: 1.0086x; 1.0086x over previous
import jax
import jax.numpy as jnp
from jax.experimental import pallas as pl
from jax.experimental.pallas import tpu as pltpu

_N = 4096
_D = 256
_K = 10
_R = 512


def _proj_kernel(x_ref, w_ref, o_ref, sq_ref):
    xw = jax.lax.dot(
        x_ref[:, :].astype(jnp.bfloat16), w_ref[:, :].astype(jnp.bfloat16),
        preferred_element_type=jnp.float32,
    )
    o_ref[:, :] = xw
    sq_ref[:, :] = jnp.sum(xw * xw, axis=1, keepdims=True)


def _knn_kernel(t_ref, xw_blk_ref, xw_ref, sqr_ref, sql_ref, idx_ref, val_ref):
    t = t_ref[0]
    xw_b = xw_ref[:, :].astype(jnp.bfloat16)
    xw_blk_b = xw_blk_ref[:, :].astype(jnp.bfloat16)
    g = jax.lax.dot_general(
        xw_blk_b, xw_b,
        (((1,), (1,)), ((), ())),
        preferred_element_type=jnp.float32,
    )
    lq = (sqr_ref[:, :] + sql_ref[:, :] - 2.0 * g) * t
    iota_f = jax.lax.broadcasted_iota(jnp.int32, (_R, _N), 1).astype(jnp.float32)
    for k in range(_K):
        m = jnp.min(lq, axis=1)
        mask = lq <= m[:, None]
        sel = jnp.where(mask, iota_f, jnp.float32(_N))
        jf = jnp.min(sel, axis=1)
        idx_ref[:, k] = jf.astype(jnp.int32)
        val_ref[:, k] = -m
        lq = jnp.where(sel <= jf[:, None], jnp.float32(jnp.inf), lq)


@jax.jit
def kernel(x, A, W, temperature):
    del A
    t = jnp.exp(jnp.clip(temperature, -5.0, 5.0)).astype(jnp.float32)

    xw, sq_col = pl.pallas_call(
        _proj_kernel,
        grid=(_N // _R,),
        in_specs=[
            pl.BlockSpec((_R, _D), lambda i: (i, 0)),
            pl.BlockSpec((_D, _D), lambda i: (0, 0)),
        ],
        out_specs=[
            pl.BlockSpec((_R, _D), lambda i: (i, 0)),
            pl.BlockSpec((_R, 1), lambda i: (i, 0)),
        ],
        out_shape=[
            jax.ShapeDtypeStruct((_N, _D), jnp.float32),
            jax.ShapeDtypeStruct((_N, 1), jnp.float32),
        ],
    )(x, W)

    idx, val = pl.pallas_call(
        _knn_kernel,
        grid=(_N // _R,),
        in_specs=[
            pl.BlockSpec(memory_space=pltpu.SMEM),
            pl.BlockSpec((_R, _D), lambda i: (i, 0)),
            pl.BlockSpec((_N, _D), lambda i: (0, 0)),
            pl.BlockSpec((_R, 1), lambda i: (i, 0)),
            pl.BlockSpec((1, _N), lambda i: (0, 0)),
        ],
        out_specs=[
            pl.BlockSpec((_R, _K), lambda i: (i, 0)),
            pl.BlockSpec((_R, _K), lambda i: (i, 0)),
        ],
        out_shape=[
            jax.ShapeDtypeStruct((_N, _K), jnp.int32),
            jax.ShapeDtypeStruct((_N, _K), jnp.float32),
        ],
    )(t.reshape(1), xw, xw, sq_col, sq_col.reshape(1, _N))

    rows = jnp.broadcast_to(
        jnp.arange(_N, dtype=jnp.int32)[:, None], (_N, _K)
    ).reshape(-1)
    edges_hat = jnp.stack((idx.reshape(-1), rows), axis=0)
    return (xw[None], edges_hat, val[None])
